# Initial kernel scaffold; baseline (speedup 1.0000x reference)
import functools

import jax
import jax.numpy as jnp
from jax import lax
from jax.experimental import pallas as pl
from jax.experimental.pallas import tpu as pltpu
from jax.experimental.pallas import tpu_sc as plsc

N = 10000
E = 320000
F = 128
DE = 16
H = 256
OUT = 128

NC = 2
NS = 16
NW = NC * NS

GF = 144
MW = 288
HALF = MW // NC

EPW = E // NW
GC = 80
GI = EPW // GC

EPT = E // NS
SC_C = 80
SI = EPT // SC_C

NPAD = 10240
RPT = NPAD // NS


def _sc_gather(nf_ext, src, dst):
    mesh = plsc.VectorSubcoreMesh(core_axis_name="c", subcore_axis_name="s")

    @functools.partial(
        pl.kernel,
        mesh=mesh,
        out_type=[
            jax.ShapeDtypeStruct((E, GF), jnp.float32),
            jax.ShapeDtypeStruct((E, GF), jnp.float32),
        ],
        scratch_types=[
            pltpu.VMEM((GC,), jnp.int32),
            pltpu.VMEM((GC,), jnp.int32),
            pltpu.VMEM((GC, GF), jnp.float32),
            pltpu.VMEM((GC, GF), jnp.float32),
            pltpu.SemaphoreType.DMA,
            pltpu.SemaphoreType.DMA,
        ],
    )
    def k(nf_hbm, src_hbm, dst_hbm, xi_hbm, xj_hbm,
          didx_v, sidx_v, xi_v, xj_v, sem_i, sem_j):
        wid = lax.axis_index("s") * NC + lax.axis_index("c")
        gbase = wid * EPW

        def body(i, carry):
            base = gbase + i * GC
            pltpu.sync_copy(dst_hbm.at[pl.ds(base, GC)], didx_v)
            pltpu.sync_copy(src_hbm.at[pl.ds(base, GC)], sidx_v)
            hi = pltpu.async_copy(nf_hbm.at[didx_v], xi_v, sem_i)
            hj = pltpu.async_copy(nf_hbm.at[sidx_v], xj_v, sem_j)
            hi.wait()
            hj.wait()
            pltpu.sync_copy(xi_v, xi_hbm.at[pl.ds(base, GC), :])
            pltpu.sync_copy(xj_v, xj_hbm.at[pl.ds(base, GC), :])
            return carry

        lax.fori_loop(0, GI, body, 0)

    return k(nf_ext, src, dst)


def _tc_mlp(xi, xj, ef, W1a, W1b, w1c, W1d, b1, W2, b2):
    BE = 512
    nb = E // BE

    def body(xi_ref, xj_ref, ef_ref, w1a_ref, w1b_ref, w1c_ref, w1d_ref,
             b1_ref, w2_ref, b2_ref, out_ref):
        x_i = xi_ref[:, :F]
        x_j = xj_ref[:, :F]
        ci = xi_ref[:, F:GF]
        cj = xj_ref[:, F:GF]
        rel = ci - cj
        d2 = jnp.sum(rel * rel, axis=1, keepdims=True)
        h = (
            jnp.dot(x_i, w1a_ref[...], preferred_element_type=jnp.float32)
            + jnp.dot(x_j, w1b_ref[...], preferred_element_type=jnp.float32)
            + jnp.dot(ef_ref[...], w1d_ref[...],
                      preferred_element_type=jnp.float32)
            + d2 * w1c_ref[...]
            + b1_ref[...]
        )
        h = jnp.where(h >= 0, h, 0.01 * h)
        m = jnp.tanh(jnp.dot(h, w2_ref[...],
                             preferred_element_type=jnp.float32) + b2_ref[...])
        out_ref[:, :H] = m
        out_ref[:, H:] = jnp.ones((BE, MW - H), jnp.float32)

    fixed = lambda i: (0, 0)
    return pl.pallas_call(
        body,
        grid=(nb,),
        in_specs=[
            pl.BlockSpec((BE, GF), lambda i: (i, 0)),
            pl.BlockSpec((BE, GF), lambda i: (i, 0)),
            pl.BlockSpec((BE, DE), lambda i: (i, 0)),
            pl.BlockSpec((F, H), fixed),
            pl.BlockSpec((F, H), fixed),
            pl.BlockSpec((1, H), fixed),
            pl.BlockSpec((DE, H), fixed),
            pl.BlockSpec((1, H), fixed),
            pl.BlockSpec((H, H), fixed),
            pl.BlockSpec((1, H), fixed),
        ],
        out_specs=pl.BlockSpec((BE, MW), lambda i: (i, 0)),
        out_shape=jax.ShapeDtypeStruct((E, MW), jnp.float32),
    )(xi, xj, ef, W1a, W1b, w1c, W1d, b1, W2, b2)


def _sc_scatter(m, dst, zrows):
    mesh = plsc.VectorSubcoreMesh(core_axis_name="c", subcore_axis_name="s")

    @functools.partial(
        pl.kernel,
        mesh=mesh,
        out_type=jax.ShapeDtypeStruct((NPAD, MW), jnp.float32),
        scratch_types=[
            pltpu.VMEM((SC_C,), jnp.int32),
            pltpu.VMEM((SC_C, HALF), jnp.float32),
            pltpu.VMEM_SHARED((NPAD, HALF), jnp.float32),
            pltpu.SemaphoreType.DMA,
        ],
    )
    def k(m_hbm, dst_hbm, z_hbm, agg_hbm, idx_v, rows_v, acc_sh, sem):
        c = lax.axis_index("c")
        s = lax.axis_index("s")
        col = c * HALF

        pltpu.sync_copy(z_hbm.at[pl.ds(s * RPT, RPT), :],
                        acc_sh.at[pl.ds(s * RPT, RPT), :])
        plsc.subcore_barrier()

        tbase = s * EPT

        def body(i, carry):
            base = tbase + i * SC_C
            pltpu.sync_copy(dst_hbm.at[pl.ds(base, SC_C)], idx_v)
            pltpu.sync_copy(m_hbm.at[pl.ds(base, SC_C), pl.ds(col, HALF)],
                            rows_v)
            pltpu.sync_copy(rows_v, acc_sh.at[idx_v], add=True)
            return carry

        lax.fori_loop(0, SI, body, 0)
        plsc.subcore_barrier()

        pltpu.sync_copy(acc_sh.at[pl.ds(s * RPT, RPT), :],
                        agg_hbm.at[pl.ds(s * RPT, RPT), pl.ds(col, HALF)])

    return k(m, dst, zrows)


def _tc_node(nf_pad, agg_ext, W_root, W_agg, b_out):
    BN = 1024
    nb = NPAD // BN

    def body(nf_ref, agg_ref, wr_ref, wa_ref, bo_ref, out_ref):
        a = agg_ref[:, :H]
        cnt = agg_ref[:, H:H + 1]
        mean_agg = a / jnp.maximum(cnt, 1.0)
        o = (
            jnp.dot(nf_ref[...], wr_ref[...],
                    preferred_element_type=jnp.float32)
            + jnp.dot(mean_agg, wa_ref[...],
                      preferred_element_type=jnp.float32)
            + bo_ref[...]
        )
        o = jnp.where(o >= 0, o, 0.01 * o)
        mu = jnp.mean(o, axis=1, keepdims=True)
        var = jnp.mean((o - mu) * (o - mu), axis=1, keepdims=True)
        out_ref[...] = (o - mu) * jax.lax.rsqrt(var + 1e-5)

    fixed = lambda i: (0, 0)
    return pl.pallas_call(
        body,
        grid=(nb,),
        in_specs=[
            pl.BlockSpec((BN, F), lambda i: (i, 0)),
            pl.BlockSpec((BN, MW), lambda i: (i, 0)),
            pl.BlockSpec((F, OUT), fixed),
            pl.BlockSpec((H, OUT), fixed),
            pl.BlockSpec((1, OUT), fixed),
        ],
        out_specs=pl.BlockSpec((BN, OUT), lambda i: (i, 0)),
        out_shape=jax.ShapeDtypeStruct((NPAD, OUT), jnp.float32),
    )(nf_pad, agg_ext, W_root, W_agg, b_out)


def kernel(coords, node_feat, edge_feat, edge_index, batch_index,
           num_sampled_nodes_per_hop, num_sampled_edges_per_hop,
           W1, b1, W2, b2, W_root, W_agg, b_out):
    src = edge_index[0]
    dst = edge_index[1]

    nf_ext = jnp.concatenate(
        [node_feat, coords, jnp.zeros((N, GF - F - 3), jnp.float32)], axis=1)

    xi, xj = _sc_gather(nf_ext, src, dst)

    W1a = W1[:F]
    W1b = W1[F:2 * F]
    w1c = W1[2 * F:2 * F + 1]
    W1d = W1[2 * F + 1:]

    m = _tc_mlp(xi, xj, edge_feat, W1a, W1b, w1c, W1d,
                b1.reshape(1, H), W2, b2.reshape(1, H))

    zrows = jnp.zeros((NPAD, HALF), jnp.float32)
    agg_ext = _sc_scatter(m, dst, zrows)

    nf_pad = jnp.concatenate(
        [node_feat, jnp.zeros((NPAD - N, F), jnp.float32)], axis=0)
    out = _tc_node(nf_pad, agg_ext, W_root, W_agg, b_out.reshape(1, OUT))

    return (coords, edge_index, out[:N])


# speedup vs baseline: 3.2557x; 3.2557x over previous
import functools

import jax
import jax.numpy as jnp
from jax import lax
from jax.experimental import pallas as pl
from jax.experimental.pallas import tpu as pltpu
from jax.experimental.pallas import tpu_sc as plsc

N = 10000
E = 320000
F = 128
DE = 16
H = 256
OUT = 128

NC = 2
NS = 16
NW = NC * NS
L = 16

MW = H
HALF = MW // NC

EPW = E // NW
GC = 80
GI = EPW // GC

EPT = E // NS
SC_C = 80
SI = EPT // SC_C

NPAD = 10240
RPT = NPAD // NS


def _sc_gather(node_feat, cx, cy, cz, src, dst):
    mesh = plsc.VectorSubcoreMesh(core_axis_name="c", subcore_axis_name="s")

    @functools.partial(
        pl.kernel,
        mesh=mesh,
        compiler_params=pltpu.CompilerParams(needs_layout_passes=False),
        out_type=[
            jax.ShapeDtypeStruct((E, F), jnp.float32),
            jax.ShapeDtypeStruct((E, F), jnp.float32),
            jax.ShapeDtypeStruct((E,), jnp.float32),
        ],
        scratch_types=[
            pltpu.VMEM((GC,), jnp.int32),
            pltpu.VMEM((GC,), jnp.int32),
            pltpu.VMEM((GC, F), jnp.float32),
            pltpu.VMEM((GC, F), jnp.float32),
            pltpu.VMEM((GC,), jnp.float32),
            pltpu.VMEM((N,), jnp.float32),
            pltpu.VMEM((N,), jnp.float32),
            pltpu.VMEM((N,), jnp.float32),
            pltpu.SemaphoreType.DMA,
            pltpu.SemaphoreType.DMA,
        ],
    )
    def k(nf_hbm, cx_hbm, cy_hbm, cz_hbm, src_hbm, dst_hbm,
          xi_hbm, xj_hbm, d2_hbm,
          didx_v, sidx_v, xi_v, xj_v, d2_v, cx_v, cy_v, cz_v, sem_i, sem_j):
        wid = lax.axis_index("s") * NC + lax.axis_index("c")
        gbase = wid * EPW

        pltpu.sync_copy(cx_hbm, cx_v)
        pltpu.sync_copy(cy_hbm, cy_v)
        pltpu.sync_copy(cz_hbm, cz_v)

        def body(i, carry):
            base = gbase + i * GC
            pltpu.sync_copy(dst_hbm.at[pl.ds(base, GC)], didx_v)
            pltpu.sync_copy(src_hbm.at[pl.ds(base, GC)], sidx_v)
            hi = pltpu.async_copy(nf_hbm.at[didx_v], xi_v, sem_i)
            hj = pltpu.async_copy(nf_hbm.at[sidx_v], xj_v, sem_j)
            for j in range(GC // L):
                di = didx_v[pl.ds(j * L, L)]
                si = sidx_v[pl.ds(j * L, L)]
                dx = plsc.load_gather(cx_v, [di]) - plsc.load_gather(cx_v, [si])
                dy = plsc.load_gather(cy_v, [di]) - plsc.load_gather(cy_v, [si])
                dz = plsc.load_gather(cz_v, [di]) - plsc.load_gather(cz_v, [si])
                d2_v[pl.ds(j * L, L)] = dx * dx + dy * dy + dz * dz
            pltpu.sync_copy(d2_v, d2_hbm.at[pl.ds(base, GC)])
            hi.wait()
            hj.wait()
            pltpu.sync_copy(xi_v, xi_hbm.at[pl.ds(base, GC), :])
            pltpu.sync_copy(xj_v, xj_hbm.at[pl.ds(base, GC), :])
            return carry

        lax.fori_loop(0, GI, body, 0)

    return k(node_feat, cx, cy, cz, src, dst)


def _tc_mlp(xi, xj, ef_ext, W1a, W1b, W1d_ext, b1, W2, b2):
    BE = 512
    nb = E // BE
    KD = DE + 1

    def body(xi_ref, xj_ref, ef_ref, w1a_ref, w1b_ref, w1d_ref,
             b1_ref, w2_ref, b2_ref, out_ref):
        h = (
            jnp.dot(xi_ref[...], w1a_ref[...],
                    preferred_element_type=jnp.float32)
            + jnp.dot(xj_ref[...], w1b_ref[...],
                      preferred_element_type=jnp.float32)
            + jnp.dot(ef_ref[...], w1d_ref[...],
                      preferred_element_type=jnp.float32)
            + b1_ref[...]
        )
        h = jnp.where(h >= 0, h, 0.01 * h)
        m = jnp.tanh(jnp.dot(h, w2_ref[...],
                             preferred_element_type=jnp.float32) + b2_ref[...])
        out_ref[...] = m

    fixed = lambda i: (0, 0)
    return pl.pallas_call(
        body,
        grid=(nb,),
        in_specs=[
            pl.BlockSpec((BE, F), lambda i: (i, 0)),
            pl.BlockSpec((BE, F), lambda i: (i, 0)),
            pl.BlockSpec((BE, KD), lambda i: (i, 0)),
            pl.BlockSpec((F, H), fixed),
            pl.BlockSpec((F, H), fixed),
            pl.BlockSpec((KD, H), fixed),
            pl.BlockSpec((1, H), fixed),
            pl.BlockSpec((H, H), fixed),
            pl.BlockSpec((1, H), fixed),
        ],
        out_specs=pl.BlockSpec((BE, MW), lambda i: (i, 0)),
        out_shape=jax.ShapeDtypeStruct((E, MW), jnp.float32),
    )(xi, xj, ef_ext, W1a, W1b, W1d_ext, b1, W2, b2)


def _sc_scatter(m, dst, zrows):
    mesh = plsc.VectorSubcoreMesh(core_axis_name="c", subcore_axis_name="s")

    @functools.partial(
        pl.kernel,
        mesh=mesh,
        compiler_params=pltpu.CompilerParams(needs_layout_passes=False),
        out_type=[
            jax.ShapeDtypeStruct((NPAD, MW), jnp.float32),
            jax.ShapeDtypeStruct((NPAD,), jnp.float32),
        ],
        scratch_types=[
            pltpu.VMEM((SC_C,), jnp.int32),
            pltpu.VMEM((SC_C, HALF), jnp.float32),
            pltpu.VMEM((NPAD,), jnp.float32),
            pltpu.VMEM((NPAD,), jnp.float32),
            pltpu.VMEM((RPT,), jnp.float32),
            pltpu.VMEM_SHARED((NPAD, HALF), jnp.float32),
            pltpu.VMEM_SHARED((NS * NPAD,), jnp.float32),
        ],
    )
    def k(m_hbm, dst_hbm, z_hbm, agg_hbm, cnt_hbm,
          idx_v, rows_v, cnt_v, red_v, out_v, acc_sh, cnt_sh):
        c = lax.axis_index("c")
        s = lax.axis_index("s")
        col = c * HALF

        pltpu.sync_copy(z_hbm.at[pl.ds(s * RPT, RPT), :],
                        acc_sh.at[pl.ds(s * RPT, RPT), :])

        def zbody(g, carry):
            cnt_v[pl.ds(g * L, L)] = jnp.zeros((L,), jnp.float32)
            return carry

        lax.fori_loop(0, NPAD // L, zbody, 0)
        plsc.subcore_barrier()

        tbase = s * EPT
        ones = jnp.ones((L,), jnp.float32)

        def body(i, carry):
            base = tbase + i * SC_C
            pltpu.sync_copy(dst_hbm.at[pl.ds(base, SC_C)], idx_v)
            pltpu.sync_copy(m_hbm.at[pl.ds(base, SC_C), pl.ds(col, HALF)],
                            rows_v)
            pltpu.sync_copy(rows_v, acc_sh.at[idx_v], add=True)
            for j in range(SC_C // L):
                di = idx_v[pl.ds(j * L, L)]
                plsc.addupdate_scatter(cnt_v, [di], ones)
            return carry

        lax.fori_loop(0, SI, body, 0)

        pltpu.sync_copy(cnt_v, cnt_sh.at[pl.ds(s * NPAD, NPAD)])
        plsc.subcore_barrier()

        for r in range(NS):
            pltpu.sync_copy(cnt_sh.at[pl.ds(r * NPAD + s * RPT, RPT)],
                            red_v.at[pl.ds(r * RPT, RPT)])
        for g in range(RPT // L):
            acc = red_v[pl.ds(g * L, L)]
            for r in range(1, NS):
                acc = acc + red_v[pl.ds(r * RPT + g * L, L)]
            out_v[pl.ds(g * L, L)] = acc

        pltpu.sync_copy(acc_sh.at[pl.ds(s * RPT, RPT), :],
                        agg_hbm.at[pl.ds(s * RPT, RPT), pl.ds(col, HALF)])

        @pl.when(c == 0)
        def _():
            pltpu.sync_copy(out_v, cnt_hbm.at[pl.ds(s * RPT, RPT)])

    return k(m, dst, zrows)


def _tc_node(nf_pad, agg, cnt, W_root, W_agg, b_out):
    BN = 1024
    nb = NPAD // BN

    def body(nf_ref, agg_ref, cnt_ref, wr_ref, wa_ref, bo_ref, out_ref):
        mean_agg = agg_ref[...] / jnp.maximum(cnt_ref[...], 1.0)
        o = (
            jnp.dot(nf_ref[...], wr_ref[...],
                    preferred_element_type=jnp.float32)
            + jnp.dot(mean_agg, wa_ref[...],
                      preferred_element_type=jnp.float32)
            + bo_ref[...]
        )
        o = jnp.where(o >= 0, o, 0.01 * o)
        mu = jnp.mean(o, axis=1, keepdims=True)
        var = jnp.mean((o - mu) * (o - mu), axis=1, keepdims=True)
        out_ref[...] = (o - mu) * jax.lax.rsqrt(var + 1e-5)

    fixed = lambda i: (0, 0)
    return pl.pallas_call(
        body,
        grid=(nb,),
        in_specs=[
            pl.BlockSpec((BN, F), lambda i: (i, 0)),
            pl.BlockSpec((BN, MW), lambda i: (i, 0)),
            pl.BlockSpec((BN, 1), lambda i: (i, 0)),
            pl.BlockSpec((F, OUT), fixed),
            pl.BlockSpec((H, OUT), fixed),
            pl.BlockSpec((1, OUT), fixed),
        ],
        out_specs=pl.BlockSpec((BN, OUT), lambda i: (i, 0)),
        out_shape=jax.ShapeDtypeStruct((NPAD, OUT), jnp.float32),
    )(nf_pad, agg, cnt, W_root, W_agg, b_out)


def kernel(coords, node_feat, edge_feat, edge_index, batch_index,
           num_sampled_nodes_per_hop, num_sampled_edges_per_hop,
           W1, b1, W2, b2, W_root, W_agg, b_out):
    src = edge_index[0]
    dst = edge_index[1]
    cx = coords[:, 0]
    cy = coords[:, 1]
    cz = coords[:, 2]

    xi, xj, d2 = _sc_gather(node_feat, cx, cy, cz, src, dst)

    W1a = W1[:F]
    W1b = W1[F:2 * F]
    w1c = W1[2 * F:2 * F + 1]
    W1d = W1[2 * F + 1:]
    W1d_ext = jnp.concatenate([W1d, w1c], axis=0)
    ef_ext = jnp.concatenate([edge_feat, d2[:, None]], axis=1)

    m = _tc_mlp(xi, xj, ef_ext, W1a, W1b, W1d_ext,
                b1.reshape(1, H), W2, b2.reshape(1, H))

    zrows = jnp.zeros((NPAD, HALF), jnp.float32)
    agg, cnt = _sc_scatter(m, dst, zrows)

    nf_pad = jnp.concatenate(
        [node_feat, jnp.zeros((NPAD - N, F), jnp.float32)], axis=0)
    out = _tc_node(nf_pad, agg, cnt.reshape(NPAD, 1),
                   W_root, W_agg, b_out.reshape(1, OUT))

    return (coords, edge_index, out[:N])


# speedup vs baseline: 3.2782x; 1.0069x over previous
import functools

import jax
import jax.numpy as jnp
from jax import lax
from jax.experimental import pallas as pl
from jax.experimental.pallas import tpu as pltpu
from jax.experimental.pallas import tpu_sc as plsc

N = 10000
E = 320000
F = 128
DE = 16
H = 256
OUT = 128

NC = 2
NS = 16
NW = NC * NS
L = 16

MW = H
HALF = MW // NC

EPW = E // NW
GC = 80
GI = EPW // GC

EPT = E // NS
SC_C = 80
SI = EPT // SC_C

NPAD = 10240
RPT = NPAD // NS


def _sc_gather(node_feat, cx, cy, cz, src, dst):
    mesh = plsc.VectorSubcoreMesh(core_axis_name="c", subcore_axis_name="s")

    @functools.partial(
        pl.kernel,
        mesh=mesh,
        compiler_params=pltpu.CompilerParams(needs_layout_passes=False),
        out_type=[
            jax.ShapeDtypeStruct((E, F), jnp.float32),
            jax.ShapeDtypeStruct((E, F), jnp.float32),
            jax.ShapeDtypeStruct((E,), jnp.float32),
        ],
        scratch_types=[
            pltpu.VMEM((GC,), jnp.int32),
            pltpu.VMEM((GC,), jnp.int32),
            pltpu.VMEM((GC, F), jnp.float32),
            pltpu.VMEM((GC, F), jnp.float32),
            pltpu.VMEM((GC,), jnp.float32),
            pltpu.VMEM((N,), jnp.float32),
            pltpu.VMEM((N,), jnp.float32),
            pltpu.VMEM((N,), jnp.float32),
            pltpu.SemaphoreType.DMA,
            pltpu.SemaphoreType.DMA,
        ],
    )
    def k(nf_hbm, cx_hbm, cy_hbm, cz_hbm, src_hbm, dst_hbm,
          xi_hbm, xj_hbm, d2_hbm,
          didx_v, sidx_v, xi_v, xj_v, d2_v, cx_v, cy_v, cz_v, sem_i, sem_j):
        wid = lax.axis_index("s") * NC + lax.axis_index("c")
        gbase = wid * EPW

        pltpu.sync_copy(cx_hbm, cx_v)
        pltpu.sync_copy(cy_hbm, cy_v)
        pltpu.sync_copy(cz_hbm, cz_v)

        def body(i, carry):
            base = gbase + i * GC
            pltpu.sync_copy(dst_hbm.at[pl.ds(base, GC)], didx_v)
            pltpu.sync_copy(src_hbm.at[pl.ds(base, GC)], sidx_v)
            hi = pltpu.async_copy(nf_hbm.at[didx_v], xi_v, sem_i)
            hj = pltpu.async_copy(nf_hbm.at[sidx_v], xj_v, sem_j)
            for j in range(GC // L):
                di = didx_v[pl.ds(j * L, L)]
                si = sidx_v[pl.ds(j * L, L)]
                dx = plsc.load_gather(cx_v, [di]) - plsc.load_gather(cx_v, [si])
                dy = plsc.load_gather(cy_v, [di]) - plsc.load_gather(cy_v, [si])
                dz = plsc.load_gather(cz_v, [di]) - plsc.load_gather(cz_v, [si])
                d2_v[pl.ds(j * L, L)] = dx * dx + dy * dy + dz * dz
            pltpu.sync_copy(d2_v, d2_hbm.at[pl.ds(base, GC)])
            hi.wait()
            hj.wait()
            pltpu.sync_copy(xi_v, xi_hbm.at[pl.ds(base, GC), :])
            pltpu.sync_copy(xj_v, xj_hbm.at[pl.ds(base, GC), :])
            return carry

        lax.fori_loop(0, GI, body, 0)

    return k(node_feat, cx, cy, cz, src, dst)


def _tc_mlp(xi, xj, ef_ext, W1a, W1b, W1d_ext, b1, W2, b2):
    BE = 512
    nb = E // BE
    KD = DE + 1

    def body(xi_ref, xj_ref, ef_ref, w1a_ref, w1b_ref, w1d_ref,
             b1_ref, w2_ref, b2_ref, out_ref):
        h = (
            jnp.dot(xi_ref[...].astype(jnp.bfloat16), w1a_ref[...],
                    preferred_element_type=jnp.float32)
            + jnp.dot(xj_ref[...].astype(jnp.bfloat16), w1b_ref[...],
                      preferred_element_type=jnp.float32)
            + jnp.dot(ef_ref[...].astype(jnp.bfloat16), w1d_ref[...],
                      preferred_element_type=jnp.float32)
            + b1_ref[...]
        )
        h = jnp.where(h >= 0, h, 0.01 * h)
        m = jnp.tanh(jnp.dot(h.astype(jnp.bfloat16), w2_ref[...],
                             preferred_element_type=jnp.float32) + b2_ref[...])
        out_ref[...] = m

    fixed = lambda i: (0, 0)
    return pl.pallas_call(
        body,
        grid=(nb,),
        in_specs=[
            pl.BlockSpec((BE, F), lambda i: (i, 0)),
            pl.BlockSpec((BE, F), lambda i: (i, 0)),
            pl.BlockSpec((BE, KD), lambda i: (i, 0)),
            pl.BlockSpec((F, H), fixed),
            pl.BlockSpec((F, H), fixed),
            pl.BlockSpec((KD, H), fixed),
            pl.BlockSpec((1, H), fixed),
            pl.BlockSpec((H, H), fixed),
            pl.BlockSpec((1, H), fixed),
        ],
        out_specs=pl.BlockSpec((BE, MW), lambda i: (i, 0)),
        out_shape=jax.ShapeDtypeStruct((E, MW), jnp.float32),
    )(xi, xj, ef_ext, W1a, W1b, W1d_ext, b1, W2, b2)


def _sc_scatter(m, dst, zrows):
    mesh = plsc.VectorSubcoreMesh(core_axis_name="c", subcore_axis_name="s")

    @functools.partial(
        pl.kernel,
        mesh=mesh,
        compiler_params=pltpu.CompilerParams(needs_layout_passes=False),
        out_type=[
            jax.ShapeDtypeStruct((NPAD, MW), jnp.float32),
            jax.ShapeDtypeStruct((NPAD,), jnp.float32),
        ],
        scratch_types=[
            pltpu.VMEM((SC_C,), jnp.int32),
            pltpu.VMEM((SC_C, HALF), jnp.float32),
            pltpu.VMEM((NPAD,), jnp.float32),
            pltpu.VMEM((NPAD,), jnp.float32),
            pltpu.VMEM((RPT,), jnp.float32),
            pltpu.VMEM_SHARED((NPAD, HALF), jnp.float32),
            pltpu.VMEM_SHARED((NS * NPAD,), jnp.float32),
        ],
    )
    def k(m_hbm, dst_hbm, z_hbm, agg_hbm, cnt_hbm,
          idx_v, rows_v, cnt_v, red_v, out_v, acc_sh, cnt_sh):
        c = lax.axis_index("c")
        s = lax.axis_index("s")
        col = c * HALF

        pltpu.sync_copy(z_hbm.at[pl.ds(s * RPT, RPT), :],
                        acc_sh.at[pl.ds(s * RPT, RPT), :])

        def zbody(g, carry):
            cnt_v[pl.ds(g * L, L)] = jnp.zeros((L,), jnp.float32)
            return carry

        lax.fori_loop(0, NPAD // L, zbody, 0)
        plsc.subcore_barrier()

        tbase = s * EPT
        ones = jnp.ones((L,), jnp.float32)

        def body(i, carry):
            base = tbase + i * SC_C
            pltpu.sync_copy(dst_hbm.at[pl.ds(base, SC_C)], idx_v)
            pltpu.sync_copy(m_hbm.at[pl.ds(base, SC_C), pl.ds(col, HALF)],
                            rows_v)
            pltpu.sync_copy(rows_v, acc_sh.at[idx_v], add=True)
            for j in range(SC_C // L):
                di = idx_v[pl.ds(j * L, L)]
                plsc.addupdate_scatter(cnt_v, [di], ones)
            return carry

        lax.fori_loop(0, SI, body, 0)

        pltpu.sync_copy(cnt_v, cnt_sh.at[pl.ds(s * NPAD, NPAD)])
        plsc.subcore_barrier()

        for r in range(NS):
            pltpu.sync_copy(cnt_sh.at[pl.ds(r * NPAD + s * RPT, RPT)],
                            red_v.at[pl.ds(r * RPT, RPT)])
        for g in range(RPT // L):
            acc = red_v[pl.ds(g * L, L)]
            for r in range(1, NS):
                acc = acc + red_v[pl.ds(r * RPT + g * L, L)]
            out_v[pl.ds(g * L, L)] = acc

        pltpu.sync_copy(acc_sh.at[pl.ds(s * RPT, RPT), :],
                        agg_hbm.at[pl.ds(s * RPT, RPT), pl.ds(col, HALF)])

        @pl.when(c == 0)
        def _():
            pltpu.sync_copy(out_v, cnt_hbm.at[pl.ds(s * RPT, RPT)])

    return k(m, dst, zrows)


def _tc_node(nf_pad, agg, cnt, W_root, W_agg, b_out):
    BN = 1024
    nb = NPAD // BN

    def body(nf_ref, agg_ref, cnt_ref, wr_ref, wa_ref, bo_ref, out_ref):
        mean_agg = agg_ref[...] / jnp.maximum(cnt_ref[...], 1.0)
        o = (
            jnp.dot(nf_ref[...], wr_ref[...],
                    preferred_element_type=jnp.float32)
            + jnp.dot(mean_agg, wa_ref[...],
                      preferred_element_type=jnp.float32)
            + bo_ref[...]
        )
        o = jnp.where(o >= 0, o, 0.01 * o)
        mu = jnp.mean(o, axis=1, keepdims=True)
        var = jnp.mean((o - mu) * (o - mu), axis=1, keepdims=True)
        out_ref[...] = (o - mu) * jax.lax.rsqrt(var + 1e-5)

    fixed = lambda i: (0, 0)
    return pl.pallas_call(
        body,
        grid=(nb,),
        in_specs=[
            pl.BlockSpec((BN, F), lambda i: (i, 0)),
            pl.BlockSpec((BN, MW), lambda i: (i, 0)),
            pl.BlockSpec((BN, 1), lambda i: (i, 0)),
            pl.BlockSpec((F, OUT), fixed),
            pl.BlockSpec((H, OUT), fixed),
            pl.BlockSpec((1, OUT), fixed),
        ],
        out_specs=pl.BlockSpec((BN, OUT), lambda i: (i, 0)),
        out_shape=jax.ShapeDtypeStruct((NPAD, OUT), jnp.float32),
    )(nf_pad, agg, cnt, W_root, W_agg, b_out)


def kernel(coords, node_feat, edge_feat, edge_index, batch_index,
           num_sampled_nodes_per_hop, num_sampled_edges_per_hop,
           W1, b1, W2, b2, W_root, W_agg, b_out):
    src = edge_index[0]
    dst = edge_index[1]
    cx = coords[:, 0]
    cy = coords[:, 1]
    cz = coords[:, 2]

    xi, xj, d2 = _sc_gather(node_feat, cx, cy, cz, src, dst)

    W1a = W1[:F]
    W1b = W1[F:2 * F]
    w1c = W1[2 * F:2 * F + 1]
    W1d = W1[2 * F + 1:]
    W1d_ext = jnp.concatenate([W1d, w1c], axis=0)
    ef_ext = jnp.concatenate([edge_feat, d2[:, None]], axis=1)

    m = _tc_mlp(xi, xj, ef_ext,
                W1a.astype(jnp.bfloat16), W1b.astype(jnp.bfloat16),
                W1d_ext.astype(jnp.bfloat16),
                b1.reshape(1, H), W2.astype(jnp.bfloat16), b2.reshape(1, H))

    zrows = jnp.zeros((NPAD, HALF), jnp.float32)
    agg, cnt = _sc_scatter(m, dst, zrows)

    nf_pad = jnp.concatenate(
        [node_feat, jnp.zeros((NPAD - N, F), jnp.float32)], axis=0)
    out = _tc_node(nf_pad, agg, cnt.reshape(NPAD, 1),
                   W_root, W_agg, b_out.reshape(1, OUT))

    return (coords, edge_index, out[:N])


# speedup vs baseline: 4.0391x; 1.2321x over previous
import functools

import jax
import jax.numpy as jnp
from jax import lax
from jax.experimental import pallas as pl
from jax.experimental.pallas import tpu as pltpu
from jax.experimental.pallas import tpu_sc as plsc

N = 10000
E = 320000
F = 128
DE = 16
H = 256
OUT = 128

NC = 2
NS = 16
NW = NC * NS
L = 16

MW = H
HALF = MW // NC

EPW = E // NW
GC = 80
GI = EPW // GC

EPT = E // NS
SC_C = 80
SI = EPT // SC_C

NPAD = 10240
RPT = NPAD // NS


def _sc_gather(node_feat, cx, cy, cz, src, dst):
    mesh = plsc.VectorSubcoreMesh(core_axis_name="c", subcore_axis_name="s")

    @functools.partial(
        pl.kernel,
        mesh=mesh,
        compiler_params=pltpu.CompilerParams(needs_layout_passes=False),
        out_type=[
            jax.ShapeDtypeStruct((E, F), jnp.float32),
            jax.ShapeDtypeStruct((E, F), jnp.float32),
            jax.ShapeDtypeStruct((E,), jnp.float32),
            jax.ShapeDtypeStruct((NC * NPAD,), jnp.float32),
        ],
        scratch_types=[
            pltpu.VMEM((GC,), jnp.int32),
            pltpu.VMEM((GC,), jnp.int32),
            pltpu.VMEM((GC, F), jnp.float32),
            pltpu.VMEM((GC, F), jnp.float32),
            pltpu.VMEM((GC,), jnp.float32),
            pltpu.VMEM((N,), jnp.float32),
            pltpu.VMEM((N,), jnp.float32),
            pltpu.VMEM((N,), jnp.float32),
            pltpu.VMEM((NPAD,), jnp.float32),
            pltpu.VMEM((NPAD,), jnp.float32),
            pltpu.VMEM((RPT,), jnp.float32),
            pltpu.VMEM_SHARED((NS * NPAD,), jnp.float32),
            pltpu.SemaphoreType.DMA,
            pltpu.SemaphoreType.DMA,
        ],
    )
    def k(nf_hbm, cx_hbm, cy_hbm, cz_hbm, src_hbm, dst_hbm,
          xi_hbm, xj_hbm, d2_hbm, cntp_hbm,
          didx_v, sidx_v, xi_v, xj_v, d2_v, cx_v, cy_v, cz_v,
          cnt_v, red_v, out_v, cnt_sh, sem_i, sem_j):
        c = lax.axis_index("c")
        s = lax.axis_index("s")
        wid = s * NC + c
        gbase = wid * EPW

        pltpu.sync_copy(cx_hbm, cx_v)
        pltpu.sync_copy(cy_hbm, cy_v)
        pltpu.sync_copy(cz_hbm, cz_v)

        def zbody(g, carry):
            cnt_v[pl.ds(g * L, L)] = jnp.zeros((L,), jnp.float32)
            return carry

        lax.fori_loop(0, NPAD // L, zbody, 0)

        ones = jnp.ones((L,), jnp.float32)

        def body(i, carry):
            base = gbase + i * GC
            pltpu.sync_copy(dst_hbm.at[pl.ds(base, GC)], didx_v)
            pltpu.sync_copy(src_hbm.at[pl.ds(base, GC)], sidx_v)
            hi = pltpu.async_copy(nf_hbm.at[didx_v], xi_v, sem_i)
            hj = pltpu.async_copy(nf_hbm.at[sidx_v], xj_v, sem_j)
            for j in range(GC // L):
                di = didx_v[pl.ds(j * L, L)]
                si = sidx_v[pl.ds(j * L, L)]
                dx = plsc.load_gather(cx_v, [di]) - plsc.load_gather(cx_v, [si])
                dy = plsc.load_gather(cy_v, [di]) - plsc.load_gather(cy_v, [si])
                dz = plsc.load_gather(cz_v, [di]) - plsc.load_gather(cz_v, [si])
                d2_v[pl.ds(j * L, L)] = dx * dx + dy * dy + dz * dz
                plsc.addupdate_scatter(cnt_v, [di], ones)
            pltpu.sync_copy(d2_v, d2_hbm.at[pl.ds(base, GC)])
            hi.wait()
            hj.wait()
            pltpu.sync_copy(xi_v, xi_hbm.at[pl.ds(base, GC), :])
            pltpu.sync_copy(xj_v, xj_hbm.at[pl.ds(base, GC), :])
            return carry

        lax.fori_loop(0, GI, body, 0)

        pltpu.sync_copy(cnt_v, cnt_sh.at[pl.ds(s * NPAD, NPAD)])
        plsc.subcore_barrier()
        for r in range(NS):
            pltpu.sync_copy(cnt_sh.at[pl.ds(r * NPAD + s * RPT, RPT)],
                            red_v.at[pl.ds(r * RPT, RPT)])
        for g in range(RPT // L):
            acc = red_v[pl.ds(g * L, L)]
            for r in range(1, NS):
                acc = acc + red_v[pl.ds(r * RPT + g * L, L)]
            out_v[pl.ds(g * L, L)] = acc
        pltpu.sync_copy(out_v, cntp_hbm.at[pl.ds(c * NPAD + s * RPT, RPT)])

    return k(node_feat, cx, cy, cz, src, dst)


def _tc_mlp(xi, xj, ef_t, W1a, W1b, W1d_ext, b1, W2, b2):
    BE = 512
    nb = E // BE
    KD = DE + 1

    def body(xi_ref, xj_ref, ef_ref, w1a_ref, w1b_ref, w1d_ref,
             b1_ref, w2_ref, b2_ref, out_ref):
        h = (
            jnp.dot(xi_ref[...].astype(jnp.bfloat16), w1a_ref[...],
                    preferred_element_type=jnp.float32)
            + jnp.dot(xj_ref[...].astype(jnp.bfloat16), w1b_ref[...],
                      preferred_element_type=jnp.float32)
            + lax.dot_general(
                ef_ref[...].astype(jnp.bfloat16), w1d_ref[...],
                dimension_numbers=(((0,), (0,)), ((), ())),
                preferred_element_type=jnp.float32)
            + b1_ref[...]
        )
        h = jnp.where(h >= 0, h, 0.01 * h)
        m = jnp.tanh(jnp.dot(h.astype(jnp.bfloat16), w2_ref[...],
                             preferred_element_type=jnp.float32) + b2_ref[...])
        out_ref[...] = m

    fixed = lambda i: (0, 0)
    return pl.pallas_call(
        body,
        grid=(nb,),
        in_specs=[
            pl.BlockSpec((BE, F), lambda i: (i, 0)),
            pl.BlockSpec((BE, F), lambda i: (i, 0)),
            pl.BlockSpec((KD, BE), lambda i: (0, i)),
            pl.BlockSpec((F, H), fixed),
            pl.BlockSpec((F, H), fixed),
            pl.BlockSpec((KD, H), fixed),
            pl.BlockSpec((1, H), fixed),
            pl.BlockSpec((H, H), fixed),
            pl.BlockSpec((1, H), fixed),
        ],
        out_specs=pl.BlockSpec((BE, MW), lambda i: (i, 0)),
        out_shape=jax.ShapeDtypeStruct((E, MW), jnp.float32),
    )(xi, xj, ef_t, W1a, W1b, W1d_ext, b1, W2, b2)


def _sc_scatter(m, dst, zrows):
    mesh = plsc.VectorSubcoreMesh(core_axis_name="c", subcore_axis_name="s")

    @functools.partial(
        pl.kernel,
        mesh=mesh,
        compiler_params=pltpu.CompilerParams(needs_layout_passes=False),
        out_type=jax.ShapeDtypeStruct((NPAD, MW), jnp.float32),
        scratch_types=[
            pltpu.VMEM((SC_C,), jnp.int32),
            pltpu.VMEM((SC_C,), jnp.int32),
            pltpu.VMEM((SC_C, HALF), jnp.float32),
            pltpu.VMEM((SC_C, HALF), jnp.float32),
            pltpu.VMEM_SHARED((NPAD, HALF), jnp.float32),
            pltpu.SemaphoreType.DMA,
            pltpu.SemaphoreType.DMA,
            pltpu.SemaphoreType.DMA,
            pltpu.SemaphoreType.DMA,
        ],
    )
    def k(m_hbm, dst_hbm, z_hbm, agg_hbm,
          idx0_v, idx1_v, rows0_v, rows1_v, acc_sh,
          sem_i0, sem_i1, sem_r0, sem_r1):
        c = lax.axis_index("c")
        s = lax.axis_index("s")
        col = c * HALF
        idx_v = (idx0_v, idx1_v)
        rows_v = (rows0_v, rows1_v)
        sem_i = (sem_i0, sem_i1)
        sem_r = (sem_r0, sem_r1)
        tbase = s * EPT

        pltpu.sync_copy(z_hbm.at[pl.ds(s * RPT, RPT), :],
                        acc_sh.at[pl.ds(s * RPT, RPT), :])
        plsc.subcore_barrier()

        def start(chunk, b):
            base = tbase + chunk * SC_C
            pltpu.async_copy(dst_hbm.at[pl.ds(base, SC_C)], idx_v[b],
                             sem_i[b])
            pltpu.async_copy(m_hbm.at[pl.ds(base, SC_C), pl.ds(col, HALF)],
                             rows_v[b], sem_r[b])

        def drain(chunk, b):
            base = tbase + chunk * SC_C
            pltpu.make_async_copy(dst_hbm.at[pl.ds(base, SC_C)], idx_v[b],
                                  sem_i[b]).wait()
            pltpu.make_async_copy(m_hbm.at[pl.ds(base, SC_C),
                                           pl.ds(col, HALF)],
                                  rows_v[b], sem_r[b]).wait()

        start(0, 0)

        def body(g, carry):
            for b in range(2):
                chunk = g * 2 + b
                drain(chunk, b)
                nxt = lax.min(chunk + 1, SI - 1)
                start(nxt, 1 - b)
                pltpu.sync_copy(rows_v[b], acc_sh.at[idx_v[b]], add=True)
            return carry

        lax.fori_loop(0, SI // 2, body, 0)
        drain(SI - 1, 0)
        plsc.subcore_barrier()

        pltpu.sync_copy(acc_sh.at[pl.ds(s * RPT, RPT), :],
                        agg_hbm.at[pl.ds(s * RPT, RPT), pl.ds(col, HALF)])

    return k(m, dst, zrows)


def _tc_node(nf_pad, agg, cnt, W_root, W_agg, b_out):
    BN = 1024
    nb = NPAD // BN

    def body(nf_ref, agg_ref, cnt_ref, wr_ref, wa_ref, bo_ref, out_ref):
        mean_agg = agg_ref[...] / jnp.maximum(cnt_ref[...], 1.0)
        o = (
            jnp.dot(nf_ref[...], wr_ref[...],
                    preferred_element_type=jnp.float32)
            + jnp.dot(mean_agg, wa_ref[...],
                      preferred_element_type=jnp.float32)
            + bo_ref[...]
        )
        o = jnp.where(o >= 0, o, 0.01 * o)
        mu = jnp.mean(o, axis=1, keepdims=True)
        var = jnp.mean((o - mu) * (o - mu), axis=1, keepdims=True)
        out_ref[...] = (o - mu) * jax.lax.rsqrt(var + 1e-5)

    fixed = lambda i: (0, 0)
    return pl.pallas_call(
        body,
        grid=(nb,),
        in_specs=[
            pl.BlockSpec((BN, F), lambda i: (i, 0)),
            pl.BlockSpec((BN, MW), lambda i: (i, 0)),
            pl.BlockSpec((BN, 1), lambda i: (i, 0)),
            pl.BlockSpec((F, OUT), fixed),
            pl.BlockSpec((H, OUT), fixed),
            pl.BlockSpec((1, OUT), fixed),
        ],
        out_specs=pl.BlockSpec((BN, OUT), lambda i: (i, 0)),
        out_shape=jax.ShapeDtypeStruct((NPAD, OUT), jnp.float32),
    )(nf_pad, agg, cnt, W_root, W_agg, b_out)


def kernel(coords, node_feat, edge_feat, edge_index, batch_index,
           num_sampled_nodes_per_hop, num_sampled_edges_per_hop,
           W1, b1, W2, b2, W_root, W_agg, b_out):
    src = edge_index[0]
    dst = edge_index[1]
    cx = coords[:, 0]
    cy = coords[:, 1]
    cz = coords[:, 2]

    xi, xj, d2, cntp = _sc_gather(node_feat, cx, cy, cz, src, dst)
    cnt = (cntp[:NPAD] + cntp[NPAD:]).reshape(NPAD, 1)

    W1a = W1[:F]
    W1b = W1[F:2 * F]
    w1c = W1[2 * F:2 * F + 1]
    W1d = W1[2 * F + 1:]
    W1d_ext = jnp.concatenate([W1d, w1c], axis=0)
    ef_t = jnp.concatenate([edge_feat.T, d2.reshape(1, E)], axis=0)

    m = _tc_mlp(xi, xj, ef_t,
                W1a.astype(jnp.bfloat16), W1b.astype(jnp.bfloat16),
                W1d_ext.astype(jnp.bfloat16),
                b1.reshape(1, H), W2.astype(jnp.bfloat16), b2.reshape(1, H))

    zrows = jnp.zeros((NPAD, HALF), jnp.float32)
    agg = _sc_scatter(m, dst, zrows)

    nf_pad = jnp.concatenate(
        [node_feat, jnp.zeros((NPAD - N, F), jnp.float32)], axis=0)
    out = _tc_node(nf_pad, agg, cnt, W_root, W_agg, b_out.reshape(1, OUT))

    return (coords, edge_index, out[:N])


# speedup vs baseline: 5.3255x; 1.3185x over previous
import functools

import jax
import jax.numpy as jnp
from jax import lax
from jax.experimental import pallas as pl
from jax.experimental.pallas import tpu as pltpu
from jax.experimental.pallas import tpu_sc as plsc

N = 10000
E = 320000
F = 128
DE = 16
H = 256
OUT = 128

NC = 2
NS = 16
NW = NC * NS
L = 16

MW = H
HALF = MW // NC

K = 5
SE = E // K

EPW = SE // NW
GC = 80
GI = EPW // GC

EPT = SE // NS
SC_C = 80
SI = EPT // SC_C

NPAD = 10240
RPT = NPAD // NS


def _sc_gather(node_feat, cx, cy, cz, src, dst):
    mesh = plsc.VectorSubcoreMesh(core_axis_name="c", subcore_axis_name="s")

    @functools.partial(
        pl.kernel,
        mesh=mesh,
        compiler_params=pltpu.CompilerParams(needs_layout_passes=False),
        out_type=[
            jax.ShapeDtypeStruct((SE, F), jnp.float32),
            jax.ShapeDtypeStruct((SE, F), jnp.float32),
            jax.ShapeDtypeStruct((SE,), jnp.float32),
            jax.ShapeDtypeStruct((NC * NPAD,), jnp.float32),
        ],
        scratch_types=[
            pltpu.VMEM((GC,), jnp.int32),
            pltpu.VMEM((GC,), jnp.int32),
            pltpu.VMEM((GC, F), jnp.float32),
            pltpu.VMEM((GC, F), jnp.float32),
            pltpu.VMEM((GC,), jnp.float32),
            pltpu.VMEM((N,), jnp.float32),
            pltpu.VMEM((N,), jnp.float32),
            pltpu.VMEM((N,), jnp.float32),
            pltpu.VMEM((NPAD,), jnp.float32),
            pltpu.VMEM((NPAD,), jnp.float32),
            pltpu.VMEM((RPT,), jnp.float32),
            pltpu.VMEM_SHARED((NS * NPAD,), jnp.float32),
            pltpu.SemaphoreType.DMA,
            pltpu.SemaphoreType.DMA,
        ],
    )
    def k(nf_hbm, cx_hbm, cy_hbm, cz_hbm, src_hbm, dst_hbm,
          xi_hbm, xj_hbm, d2_hbm, cntp_hbm,
          didx_v, sidx_v, xi_v, xj_v, d2_v, cx_v, cy_v, cz_v,
          cnt_v, red_v, out_v, cnt_sh, sem_i, sem_j):
        c = lax.axis_index("c")
        s = lax.axis_index("s")
        wid = s * NC + c
        gbase = wid * EPW

        pltpu.sync_copy(cx_hbm, cx_v)
        pltpu.sync_copy(cy_hbm, cy_v)
        pltpu.sync_copy(cz_hbm, cz_v)

        def zbody(g, carry):
            cnt_v[pl.ds(g * L, L)] = jnp.zeros((L,), jnp.float32)
            return carry

        lax.fori_loop(0, NPAD // L, zbody, 0)

        ones = jnp.ones((L,), jnp.float32)

        def body(i, carry):
            base = gbase + i * GC
            pltpu.sync_copy(dst_hbm.at[pl.ds(base, GC)], didx_v)
            pltpu.sync_copy(src_hbm.at[pl.ds(base, GC)], sidx_v)
            hi = pltpu.async_copy(nf_hbm.at[didx_v], xi_v, sem_i)
            hj = pltpu.async_copy(nf_hbm.at[sidx_v], xj_v, sem_j)
            for j in range(GC // L):
                di = didx_v[pl.ds(j * L, L)]
                si = sidx_v[pl.ds(j * L, L)]
                dx = plsc.load_gather(cx_v, [di]) - plsc.load_gather(cx_v, [si])
                dy = plsc.load_gather(cy_v, [di]) - plsc.load_gather(cy_v, [si])
                dz = plsc.load_gather(cz_v, [di]) - plsc.load_gather(cz_v, [si])
                d2_v[pl.ds(j * L, L)] = dx * dx + dy * dy + dz * dz
                plsc.addupdate_scatter(cnt_v, [di], ones)
            pltpu.sync_copy(d2_v, d2_hbm.at[pl.ds(base, GC)])
            hi.wait()
            hj.wait()
            pltpu.sync_copy(xi_v, xi_hbm.at[pl.ds(base, GC), :])
            pltpu.sync_copy(xj_v, xj_hbm.at[pl.ds(base, GC), :])
            return carry

        lax.fori_loop(0, GI, body, 0)

        pltpu.sync_copy(cnt_v, cnt_sh.at[pl.ds(s * NPAD, NPAD)])
        plsc.subcore_barrier()
        for r in range(NS):
            pltpu.sync_copy(cnt_sh.at[pl.ds(r * NPAD + s * RPT, RPT)],
                            red_v.at[pl.ds(r * RPT, RPT)])
        for g in range(RPT // L):
            acc = red_v[pl.ds(g * L, L)]
            for r in range(1, NS):
                acc = acc + red_v[pl.ds(r * RPT + g * L, L)]
            out_v[pl.ds(g * L, L)] = acc
        pltpu.sync_copy(out_v, cntp_hbm.at[pl.ds(c * NPAD + s * RPT, RPT)])

    return k(node_feat, cx, cy, cz, src, dst)


def _tc_mlp(xi, xj, ef_t, W1a, W1b, W1d_ext, b1, W2, b2):
    BE = 512
    nb = SE // BE
    KD = DE + 1

    def body(xi_ref, xj_ref, ef_ref, w1a_ref, w1b_ref, w1d_ref,
             b1_ref, w2_ref, b2_ref, out_ref):
        h = (
            jnp.dot(xi_ref[...].astype(jnp.bfloat16), w1a_ref[...],
                    preferred_element_type=jnp.float32)
            + jnp.dot(xj_ref[...].astype(jnp.bfloat16), w1b_ref[...],
                      preferred_element_type=jnp.float32)
            + lax.dot_general(
                ef_ref[...].astype(jnp.bfloat16), w1d_ref[...],
                dimension_numbers=(((0,), (0,)), ((), ())),
                preferred_element_type=jnp.float32)
            + b1_ref[...]
        )
        h = jnp.where(h >= 0, h, 0.01 * h)
        m = jnp.tanh(jnp.dot(h.astype(jnp.bfloat16), w2_ref[...],
                             preferred_element_type=jnp.float32) + b2_ref[...])
        out_ref[...] = m

    fixed = lambda i: (0, 0)
    return pl.pallas_call(
        body,
        grid=(nb,),
        in_specs=[
            pl.BlockSpec((BE, F), lambda i: (i, 0)),
            pl.BlockSpec((BE, F), lambda i: (i, 0)),
            pl.BlockSpec((KD, BE), lambda i: (0, i)),
            pl.BlockSpec((F, H), fixed),
            pl.BlockSpec((F, H), fixed),
            pl.BlockSpec((KD, H), fixed),
            pl.BlockSpec((1, H), fixed),
            pl.BlockSpec((H, H), fixed),
            pl.BlockSpec((1, H), fixed),
        ],
        out_specs=pl.BlockSpec((BE, MW), lambda i: (i, 0)),
        out_shape=jax.ShapeDtypeStruct((SE, MW), jnp.float32),
    )(xi, xj, ef_t, W1a, W1b, W1d_ext, b1, W2, b2)


def _sc_scatter(m, dst, zrows):
    mesh = plsc.VectorSubcoreMesh(core_axis_name="c", subcore_axis_name="s")

    @functools.partial(
        pl.kernel,
        mesh=mesh,
        compiler_params=pltpu.CompilerParams(needs_layout_passes=False),
        out_type=jax.ShapeDtypeStruct((NPAD, MW), jnp.float32),
        scratch_types=[
            pltpu.VMEM((SC_C,), jnp.int32),
            pltpu.VMEM((SC_C,), jnp.int32),
            pltpu.VMEM((SC_C, HALF), jnp.float32),
            pltpu.VMEM((SC_C, HALF), jnp.float32),
            pltpu.VMEM_SHARED((NPAD, HALF), jnp.float32),
            pltpu.SemaphoreType.DMA,
            pltpu.SemaphoreType.DMA,
            pltpu.SemaphoreType.DMA,
            pltpu.SemaphoreType.DMA,
        ],
    )
    def k(m_hbm, dst_hbm, z_hbm, agg_hbm,
          idx0_v, idx1_v, rows0_v, rows1_v, acc_sh,
          sem_i0, sem_i1, sem_r0, sem_r1):
        c = lax.axis_index("c")
        s = lax.axis_index("s")
        col = c * HALF
        idx_v = (idx0_v, idx1_v)
        rows_v = (rows0_v, rows1_v)
        sem_i = (sem_i0, sem_i1)
        sem_r = (sem_r0, sem_r1)
        tbase = s * EPT

        pltpu.sync_copy(z_hbm.at[pl.ds(s * RPT, RPT), :],
                        acc_sh.at[pl.ds(s * RPT, RPT), :])
        plsc.subcore_barrier()

        def start(chunk, b):
            base = tbase + chunk * SC_C
            pltpu.async_copy(dst_hbm.at[pl.ds(base, SC_C)], idx_v[b],
                             sem_i[b])
            pltpu.async_copy(m_hbm.at[pl.ds(base, SC_C), pl.ds(col, HALF)],
                             rows_v[b], sem_r[b])

        def drain(chunk, b):
            base = tbase + chunk * SC_C
            pltpu.make_async_copy(dst_hbm.at[pl.ds(base, SC_C)], idx_v[b],
                                  sem_i[b]).wait()
            pltpu.make_async_copy(m_hbm.at[pl.ds(base, SC_C),
                                           pl.ds(col, HALF)],
                                  rows_v[b], sem_r[b]).wait()

        start(0, 0)

        def body(g, carry):
            for b in range(2):
                chunk = g * 2 + b
                drain(chunk, b)
                nxt = lax.min(chunk + 1, SI - 1)
                start(nxt, 1 - b)
                pltpu.sync_copy(rows_v[b], acc_sh.at[idx_v[b]], add=True)
            return carry

        lax.fori_loop(0, SI // 2, body, 0)
        drain(SI - 1, 0)
        plsc.subcore_barrier()

        pltpu.sync_copy(acc_sh.at[pl.ds(s * RPT, RPT), :],
                        agg_hbm.at[pl.ds(s * RPT, RPT), pl.ds(col, HALF)])

    return k(m, dst, zrows)


def _tc_node(nf_pad, aggs, cnt, W_root, W_agg, b_out):
    BN = 1024
    nb = NPAD // BN

    def body(nf_ref, a0, a1, a2, a3, a4, cnt_ref, wr_ref, wa_ref, bo_ref,
             out_ref):
        agg = a0[...] + a1[...] + a2[...] + a3[...] + a4[...]
        mean_agg = agg / jnp.maximum(cnt_ref[...], 1.0)
        o = (
            jnp.dot(nf_ref[...], wr_ref[...],
                    preferred_element_type=jnp.float32)
            + jnp.dot(mean_agg, wa_ref[...],
                      preferred_element_type=jnp.float32)
            + bo_ref[...]
        )
        o = jnp.where(o >= 0, o, 0.01 * o)
        mu = jnp.mean(o, axis=1, keepdims=True)
        var = jnp.mean((o - mu) * (o - mu), axis=1, keepdims=True)
        out_ref[...] = (o - mu) * jax.lax.rsqrt(var + 1e-5)

    fixed = lambda i: (0, 0)
    agg_spec = pl.BlockSpec((BN, MW), lambda i: (i, 0))
    return pl.pallas_call(
        body,
        grid=(nb,),
        in_specs=[
            pl.BlockSpec((BN, F), lambda i: (i, 0)),
            agg_spec, agg_spec, agg_spec, agg_spec, agg_spec,
            pl.BlockSpec((BN, 1), lambda i: (i, 0)),
            pl.BlockSpec((F, OUT), fixed),
            pl.BlockSpec((H, OUT), fixed),
            pl.BlockSpec((1, OUT), fixed),
        ],
        out_specs=pl.BlockSpec((BN, OUT), lambda i: (i, 0)),
        out_shape=jax.ShapeDtypeStruct((NPAD, OUT), jnp.float32),
    )(nf_pad, *aggs, cnt, W_root, W_agg, b_out)


def kernel(coords, node_feat, edge_feat, edge_index, batch_index,
           num_sampled_nodes_per_hop, num_sampled_edges_per_hop,
           W1, b1, W2, b2, W_root, W_agg, b_out):
    src = edge_index[0]
    dst = edge_index[1]
    cx = coords[:, 0]
    cy = coords[:, 1]
    cz = coords[:, 2]

    W1a = W1[:F].astype(jnp.bfloat16)
    W1b = W1[F:2 * F].astype(jnp.bfloat16)
    w1c = W1[2 * F:2 * F + 1]
    W1d = W1[2 * F + 1:]
    W1d_ext = jnp.concatenate([W1d, w1c], axis=0).astype(jnp.bfloat16)
    W2b = W2.astype(jnp.bfloat16)
    b1r = b1.reshape(1, H)
    b2r = b2.reshape(1, H)
    ef_t_full = edge_feat.T

    zrows = jnp.zeros((NPAD, HALF), jnp.float32)

    aggs = []
    cnt_total = jnp.zeros((NPAD,), jnp.float32)
    for k in range(K):
        sl = slice(k * SE, (k + 1) * SE)
        xi, xj, d2, cntp = _sc_gather(node_feat, cx, cy, cz,
                                      src[sl], dst[sl])
        cnt_total = cnt_total + cntp[:NPAD] + cntp[NPAD:]
        ef_t = jnp.concatenate([ef_t_full[:, sl], d2.reshape(1, SE)], axis=0)
        m = _tc_mlp(xi, xj, ef_t, W1a, W1b, W1d_ext, b1r, W2b, b2r)
        aggs.append(_sc_scatter(m, dst[sl], zrows))

    nf_pad = jnp.concatenate(
        [node_feat, jnp.zeros((NPAD - N, F), jnp.float32)], axis=0)
    out = _tc_node(nf_pad, aggs, cnt_total.reshape(NPAD, 1),
                   W_root, W_agg, b_out.reshape(1, OUT))

    return (coords, edge_index, out[:N])
